# baseline (speedup 1.0000x reference)
import jax
import jax.numpy as jnp
from jax.experimental import pallas as pl
from jax.experimental.pallas import tpu as pltpu


def _pick_tile(n, cap=400):
    best = 8
    for t in range(8, cap + 1, 8):
        if n % t == 0:
            best = t
    return best


def _sums_kernel(t_ref, row_ref, col_ref):
    i = pl.program_id(0)
    blk = t_ref[...]
    row_ref[...] = jnp.sum(blk, axis=1, keepdims=True)
    part = jnp.sum(blk, axis=0, keepdims=True)

    @pl.when(i == 0)
    def _():
        col_ref[...] = part

    @pl.when(i > 0)
    def _():
        col_ref[...] = col_ref[...] + part


def _hs_kernel(x_ref, w1t_ref, b1_ref, d1_ref, d2_ref, hs_ref, s_ref):
    s = jnp.sqrt(d1_ref[...] * d2_ref[...])
    h = jnp.dot(x_ref[...], w1t_ref[...], preferred_element_type=jnp.float32)
    hs_ref[...] = s * (h + b1_ref[...])
    s_ref[...] = s


def _spmm1_kernel(t_ref, hs_ref, w2t_ref, b2_ref, s_ref, z_ref):
    t = jnp.dot(t_ref[...], hs_ref[...], preferred_element_type=jnp.float32)
    r = jnp.maximum(t, 0.0)
    z = jnp.dot(r, w2t_ref[...], preferred_element_type=jnp.float32) + b2_ref[...]
    z_ref[...] = z * s_ref[...]


def _spmm2_kernel(t_ref, z_ref, o_ref):
    o_ref[...] = jnp.dot(t_ref[...], z_ref[...], preferred_element_type=jnp.float32)


def kernel(X, tilde, W1, b1, W2, b2):
    n, feat = X.shape
    hid = W1.shape[0]
    ncls = W2.shape[0]
    ti = _pick_tile(n)
    nb = n // ti

    row, col = pl.pallas_call(
        _sums_kernel,
        grid=(nb,),
        in_specs=[pl.BlockSpec((ti, n), lambda i: (i, 0))],
        out_specs=[
            pl.BlockSpec((ti, 1), lambda i: (i, 0)),
            pl.BlockSpec((1, n), lambda i: (0, 0)),
        ],
        out_shape=[
            jax.ShapeDtypeStruct((n, 1), jnp.float32),
            jax.ShapeDtypeStruct((1, n), jnp.float32),
        ],
        compiler_params=pltpu.CompilerParams(
            dimension_semantics=("arbitrary",),
        ),
    )(tilde)

    d1 = col.reshape(n, 1)

    hs, s = pl.pallas_call(
        _hs_kernel,
        out_shape=[
            jax.ShapeDtypeStruct((n, hid), jnp.float32),
            jax.ShapeDtypeStruct((n, 1), jnp.float32),
        ],
    )(X, W1.T, b1.reshape(1, hid), d1, row)

    z = pl.pallas_call(
        _spmm1_kernel,
        grid=(nb,),
        in_specs=[
            pl.BlockSpec((ti, n), lambda i: (i, 0)),
            pl.BlockSpec((n, hid), lambda i: (0, 0)),
            pl.BlockSpec((hid, ncls), lambda i: (0, 0)),
            pl.BlockSpec((1, ncls), lambda i: (0, 0)),
            pl.BlockSpec((ti, 1), lambda i: (i, 0)),
        ],
        out_specs=pl.BlockSpec((ti, ncls), lambda i: (i, 0)),
        out_shape=jax.ShapeDtypeStruct((n, ncls), jnp.float32),
        compiler_params=pltpu.CompilerParams(
            dimension_semantics=("arbitrary",),
        ),
    )(tilde, hs, W2.T, b2.reshape(1, ncls), s)

    o = pl.pallas_call(
        _spmm2_kernel,
        grid=(nb,),
        in_specs=[
            pl.BlockSpec((ti, n), lambda i: (i, 0)),
            pl.BlockSpec((n, ncls), lambda i: (0, 0)),
        ],
        out_specs=pl.BlockSpec((ti, ncls), lambda i: (i, 0)),
        out_shape=jax.ShapeDtypeStruct((n, ncls), jnp.float32),
        compiler_params=pltpu.CompilerParams(
            dimension_semantics=("arbitrary",),
        ),
    )(tilde, z)
    return o
